# baseline (speedup 1.0000x reference)
import jax
import jax.numpy as jnp
from jax import lax
from jax.experimental import pallas as pl
from jax.experimental.pallas import tpu as pltpu, tpu_sc as plsc

B = 4096
LU = 20
LM = 50
E = 12
EP = 16
NC, NS = 2, 16
NW = NC * NS
BPW = B // NW
CB = 8
NCHUNK = BPW // CB
U_ROWS = CB * LU
M_ROWS = CB * LM
OUT_W = CB * LU * LM


def _body(user_r, media_r, ut_hbm, mt_hbm, out_hbm,
          idx_u, idx_m, ue_rows, me_rows, out_buf, sem):
    wid = lax.axis_index("s") * NC + lax.axis_index("c")
    iota = lax.iota(jnp.int32, 16)
    ecols = [jnp.broadcast_to(jnp.int32(e), (16,)) for e in range(E)]
    tail_mask = iota < 2

    def chunk(c, carry):
        ur0 = wid * (BPW * LU // 80) + c * (U_ROWS // 80)
        mr0 = wid * (BPW * LM // 100) + c * (M_ROWS // 100)
        pltpu.sync_copy(user_r.at[pl.ds(ur0, 2)], idx_u)
        pltpu.sync_copy(media_r.at[pl.ds(mr0, 4)], idx_m)

        cps = []
        for r in range(2):
            cps.append(pltpu.async_copy(
                ut_hbm.at[idx_u.at[r]],
                ue_rows.at[pl.ds(r * 80, 80)], sem))
        for r in range(4):
            cps.append(pltpu.async_copy(
                mt_hbm.at[idx_m.at[r]],
                me_rows.at[pl.ds(r * 100, 100)], sem))
        for cp in cps:
            cp.wait()

        @plsc.parallel_loop(0, CB)
        def batch(b):
            ub = b * LU
            mb = b * LM
            for ib in range(2):
                accs = [[None] * 4 for _ in range(10)]
                for e in range(E):
                    mv = [plsc.load_gather(
                              me_rows, [iota + (mb + jc * 16), ecols[e]])
                          for jc in range(4)]
                    for ii in range(10):
                        i = ib * 10 + ii
                        s = plsc.load_gather(
                            ue_rows,
                            [jnp.broadcast_to(ub + i, (16,)), ecols[e]])
                        for jc in range(4):
                            p = s * mv[jc]
                            accs[ii][jc] = p if e == 0 else accs[ii][jc] + p
                for ii in range(10):
                    i = ib * 10 + ii
                    base = b * (LU * LM) + i * LM
                    for jc in range(3):
                        v = 1.0 / (1.0 + jnp.exp(-accs[ii][jc]))
                        out_buf[pl.ds(base + jc * 16, 16)] = v
                    v = 1.0 / (1.0 + jnp.exp(-accs[ii][3]))
                    plsc.store_scatter(out_buf, [iota + (base + 48)], v,
                                       mask=tail_mask)

        base = (wid * NCHUNK + c) * OUT_W
        pltpu.sync_copy(out_buf.at[pl.ds(0, OUT_W)],
                        out_hbm.at[pl.ds(base, OUT_W)])
        return carry

    lax.fori_loop(0, NCHUNK, chunk, 0)


@jax.jit
def kernel(user, media, user_table, media_table):
    user_r = user.astype(jnp.int32).reshape(B * LU // 80, 80)
    media_r = media.astype(jnp.int32).reshape(B * LM // 100, 100)
    mesh = plsc.VectorSubcoreMesh(core_axis_name="c", subcore_axis_name="s",
                                  num_cores=NC, num_subcores=NS)
    out = pl.kernel(
        _body,
        out_type=jax.ShapeDtypeStruct((B * LU * LM,), jnp.float32),
        mesh=mesh,
        scratch_types=[
            pltpu.VMEM((2, 80), jnp.int32),
            pltpu.VMEM((4, 100), jnp.int32),
            pltpu.VMEM((U_ROWS, EP), jnp.float32),
            pltpu.VMEM((M_ROWS + 16, EP), jnp.float32),
            pltpu.VMEM((OUT_W + 16,), jnp.float32),
            pltpu.SemaphoreType.DMA,
        ],
        compiler_params=pltpu.CompilerParams(needs_layout_passes=False,
                                             use_tc_tiling_on_sc=False,
                                             disable_bounds_checks=True),
    )(user_r, media_r,
      jnp.pad(user_table, ((0, 0), (0, EP - E))),
      jnp.pad(media_table, ((0, 0), (0, EP - E))))
    return out.reshape(B, LU, LM)


# speedup vs baseline: 1.0853x; 1.0853x over previous
import jax
import jax.numpy as jnp
from jax import lax
from jax.experimental import pallas as pl
from jax.experimental.pallas import tpu as pltpu, tpu_sc as plsc

B = 4096
LU = 20
LM = 50
E = 12
EP = 16
NC, NS = 2, 16
NW = NC * NS
BPW = B // NW
CB = 8
NCHUNK = BPW // CB
U_ROWS = CB * LU
M_ROWS = CB * LM
OUT_W = CB * LU * LM


def _body(user_r, media_r, ut_hbm, mt_hbm, out_hbm,
          idx_u, idx_m, ue_rows, me_rows, out_buf,
          sem_u, sem_m0, sem_m1, sem_m2, sem_m3):
    wid = lax.axis_index("s") * NC + lax.axis_index("c")
    iota = lax.iota(jnp.int32, 16)
    ecols = [jnp.broadcast_to(jnp.int32(e), (16,)) for e in range(E)]
    sems_m = [sem_m0, sem_m1, sem_m2, sem_m3]

    def chunk(c, carry):
        ur0 = wid * (BPW * LU // 80) + c * (U_ROWS // 80)
        mr0 = wid * (BPW * LM // 100) + c * (M_ROWS // 100)
        pltpu.sync_copy(user_r.at[pl.ds(ur0, 2)], idx_u)
        pltpu.sync_copy(media_r.at[pl.ds(mr0, 4)], idx_m)

        for r in range(2):
            pltpu.async_copy(ut_hbm.at[idx_u.at[r]],
                             ue_rows.at[pl.ds(r * 80, 80)], sem_u)
        for r in range(4):
            pltpu.async_copy(mt_hbm.at[idx_m.at[r]],
                             me_rows.at[pl.ds(r * 100, 100)], sems_m[r])

        def batch(b, carry2):
            @pl.when(b == 0)
            def _():
                pltpu.make_async_copy(ut_hbm.at[pl.ds(0, U_ROWS)],
                                      ue_rows, sem_u).wait()
            for r in range(4):
                @pl.when(b == 2 * r)
                def _(r=r):
                    pltpu.make_async_copy(
                        mt_hbm.at[pl.ds(0, 100)],
                        me_rows.at[pl.ds(0, 100)], sems_m[r]).wait()
            ub = b * LU
            mb = b * LM
            for ib in range(2):
                accs = [[None] * 4 for _ in range(10)]
                for e in range(E):
                    mv = [plsc.load_gather(
                              me_rows, [iota + (mb + jc * 16), ecols[e]])
                          for jc in range(4)]
                    for ii in range(10):
                        i = ib * 10 + ii
                        s = plsc.load_gather(
                            ue_rows,
                            [jnp.broadcast_to(ub + i, (16,)), ecols[e]])
                        for jc in range(4):
                            p = s * mv[jc]
                            accs[ii][jc] = p if e == 0 else accs[ii][jc] + p
                for ii in range(10):
                    i = ib * 10 + ii
                    for jc in range(4):
                        v = 1.0 / (1.0 + jnp.exp(-accs[ii][jc]))
                        out_buf[pl.ds(b * (LU * LM) + i * LM + jc * 16, 16)] = v
            return carry2

        lax.fori_loop(0, CB, batch, 0)

        base = (wid * NCHUNK + c) * OUT_W
        pltpu.sync_copy(out_buf.at[pl.ds(0, OUT_W)],
                        out_hbm.at[pl.ds(base, OUT_W)])
        return carry

    lax.fori_loop(0, NCHUNK, chunk, 0)


@jax.jit
def kernel(user, media, user_table, media_table):
    user_r = user.astype(jnp.int32).reshape(B * LU // 80, 80)
    media_r = media.astype(jnp.int32).reshape(B * LM // 100, 100)
    mesh = plsc.VectorSubcoreMesh(core_axis_name="c", subcore_axis_name="s",
                                  num_cores=NC, num_subcores=NS)
    out = pl.kernel(
        _body,
        out_type=jax.ShapeDtypeStruct((B * LU * LM,), jnp.float32),
        mesh=mesh,
        scratch_types=[
            pltpu.VMEM((2, 80), jnp.int32),
            pltpu.VMEM((4, 100), jnp.int32),
            pltpu.VMEM((U_ROWS, EP), jnp.float32),
            pltpu.VMEM((M_ROWS + 16, EP), jnp.float32),
            pltpu.VMEM((OUT_W + 16,), jnp.float32),
            pltpu.SemaphoreType.DMA,
            pltpu.SemaphoreType.DMA,
            pltpu.SemaphoreType.DMA,
            pltpu.SemaphoreType.DMA,
            pltpu.SemaphoreType.DMA,
        ],
        compiler_params=pltpu.CompilerParams(needs_layout_passes=False,
                                             use_tc_tiling_on_sc=False,
                                             disable_bounds_checks=True),
    )(user_r, media_r,
      jnp.pad(user_table, ((0, 0), (0, EP - E))),
      jnp.pad(media_table, ((0, 0), (0, EP - E))))
    return out.reshape(B, LU, LM)


# speedup vs baseline: 1.0973x; 1.0111x over previous
import jax
import jax.numpy as jnp
from jax import lax
from jax.experimental import pallas as pl
from jax.experimental.pallas import tpu as pltpu, tpu_sc as plsc

B = 4096
LU = 20
LM = 50
E = 12
EP = 16
NC, NS = 2, 16
NW = NC * NS
BPW = B // NW
CB = 8
NCHUNK = BPW // CB
U_ROWS = CB * LU
M_ROWS = CB * LM
OUT_W = CB * LU * LM


def _body(user_r, media_r, ut_hbm, mt_hbm, out_hbm,
          idx_u, idx_m, ue_rows, me_rows, out_buf,
          sem_u, sem_m0, sem_m1, sem_m2, sem_m3, sem_o):
    wid = lax.axis_index("s") * NC + lax.axis_index("c")
    iota = lax.iota(jnp.int32, 16)
    ecols = [jnp.broadcast_to(jnp.int32(e), (16,)) for e in range(E)]
    sems_m = [sem_m0, sem_m1, sem_m2, sem_m3]

    def chunk(c, carry):
        ur0 = wid * (BPW * LU // 80) + c * (U_ROWS // 80)
        mr0 = wid * (BPW * LM // 100) + c * (M_ROWS // 100)
        pltpu.sync_copy(user_r.at[pl.ds(ur0, 2)], idx_u)
        pltpu.sync_copy(media_r.at[pl.ds(mr0, 4)], idx_m)

        for r in range(2):
            pltpu.async_copy(ut_hbm.at[idx_u.at[r]],
                             ue_rows.at[pl.ds(r * 80, 80)], sem_u)
        for r in range(4):
            pltpu.async_copy(mt_hbm.at[idx_m.at[r]],
                             me_rows.at[pl.ds(r * 100, 100)], sems_m[r])

        def batch(b, carry2):
            @pl.when(b == 0)
            def _():
                pltpu.make_async_copy(ut_hbm.at[pl.ds(0, U_ROWS)],
                                      ue_rows, sem_u).wait()

            @pl.when(jnp.logical_and(b == 0, c > 0))
            def _():
                pltpu.make_async_copy(out_buf.at[pl.ds(0, OUT_W)],
                                      out_hbm.at[pl.ds(0, OUT_W)],
                                      sem_o).wait()
            for r in range(4):
                @pl.when(b == 2 * r)
                def _(r=r):
                    pltpu.make_async_copy(
                        mt_hbm.at[pl.ds(0, 100)],
                        me_rows.at[pl.ds(0, 100)], sems_m[r]).wait()
            ub = b * LU
            mb = b * LM
            for ib in range(2):
                accs = [[None] * 4 for _ in range(10)]
                for e in range(E):
                    mv = [plsc.load_gather(
                              me_rows, [iota + (mb + jc * 16), ecols[e]])
                          for jc in range(4)]
                    for ii in range(10):
                        i = ib * 10 + ii
                        s = plsc.load_gather(
                            ue_rows,
                            [jnp.broadcast_to(ub + i, (16,)), ecols[e]])
                        for jc in range(4):
                            p = s * mv[jc]
                            accs[ii][jc] = p if e == 0 else accs[ii][jc] + p
                for ii in range(10):
                    i = ib * 10 + ii
                    for jc in range(4):
                        v = 1.0 / (1.0 + jnp.exp(-accs[ii][jc]))
                        out_buf[pl.ds(b * (LU * LM) + i * LM + jc * 16, 16)] = v
            return carry2

        lax.fori_loop(0, CB, batch, 0)

        base = (wid * NCHUNK + c) * OUT_W
        pltpu.async_copy(out_buf.at[pl.ds(0, OUT_W)],
                         out_hbm.at[pl.ds(base, OUT_W)], sem_o)
        return carry

    lax.fori_loop(0, NCHUNK, chunk, 0)
    pltpu.make_async_copy(out_buf.at[pl.ds(0, OUT_W)],
                          out_hbm.at[pl.ds(0, OUT_W)], sem_o).wait()


@jax.jit
def kernel(user, media, user_table, media_table):
    user_r = user.astype(jnp.int32).reshape(B * LU // 80, 80)
    media_r = media.astype(jnp.int32).reshape(B * LM // 100, 100)
    mesh = plsc.VectorSubcoreMesh(core_axis_name="c", subcore_axis_name="s",
                                  num_cores=NC, num_subcores=NS)
    out = pl.kernel(
        _body,
        out_type=jax.ShapeDtypeStruct((B * LU * LM,), jnp.float32),
        mesh=mesh,
        scratch_types=[
            pltpu.VMEM((2, 80), jnp.int32),
            pltpu.VMEM((4, 100), jnp.int32),
            pltpu.VMEM((U_ROWS, EP), jnp.float32),
            pltpu.VMEM((M_ROWS + 16, EP), jnp.float32),
            pltpu.VMEM((OUT_W + 16,), jnp.float32),
            pltpu.SemaphoreType.DMA,
            pltpu.SemaphoreType.DMA,
            pltpu.SemaphoreType.DMA,
            pltpu.SemaphoreType.DMA,
            pltpu.SemaphoreType.DMA,
            pltpu.SemaphoreType.DMA,
        ],
        compiler_params=pltpu.CompilerParams(needs_layout_passes=False,
                                             use_tc_tiling_on_sc=False,
                                             disable_bounds_checks=True),
    )(user_r, media_r,
      jnp.pad(user_table, ((0, 0), (0, EP - E))),
      jnp.pad(media_table, ((0, 0), (0, EP - E))))
    return out.reshape(B, LU, LM)
